# Initial kernel scaffold; baseline (speedup 1.0000x reference)
import jax
import jax.numpy as jnp
from jax.experimental import pallas as pl


def kernel(hidden_states, ln1_g, ln1_b, W_attn, b_attn, W_o, b_o, ln2_g, ln2_b, Wg, Wfc, bfc, Wpj, bpj):
    raise NotImplementedError("write your pallas kernel here")


# speedup vs baseline: 1.0114x; 1.0114x over previous
import functools

import jax
import jax.numpy as jnp
import numpy as np
from jax.experimental import pallas as pl
from jax.experimental.pallas import tpu as pltpu

S = 2048
D = 768
H = 12
HD = D // H
E = 8
TOPK = 2
INNER = 3072
EPS = 1e-5

BT = 256
BQ = 256
GT = 128
A = S * TOPK
NT = A // GT + (E - 1)
P = NT * GT


def _ln(x, g, b):
    m = jnp.mean(x, axis=-1, keepdims=True)
    xc = x - m
    v = jnp.mean(xc * xc, axis=-1, keepdims=True)
    return xc * jax.lax.rsqrt(v + EPS) * g + b


def _gelu_new(x):
    c = np.sqrt(2.0 / np.pi).astype(np.float32)
    return 0.5 * x * (1.0 + jnp.tanh(c * (x + 0.044715 * x * x * x)))


def _k1_body(x_ref, g_ref, b_ref, w_ref, bias_ref, qkv_ref):
    h = _ln(x_ref[...], g_ref[...], b_ref[...])
    qkv_ref[...] = jnp.dot(h, w_ref[...], preferred_element_type=jnp.float32) + bias_ref[...]


def _k1(x, g, b, w, bias):
    return pl.pallas_call(
        _k1_body,
        grid=(S // BT,),
        in_specs=[
            pl.BlockSpec((BT, D), lambda i: (i, 0)),
            pl.BlockSpec((D,), lambda i: (0,)),
            pl.BlockSpec((D,), lambda i: (0,)),
            pl.BlockSpec((D, 3 * D), lambda i: (0, 0)),
            pl.BlockSpec((3 * D,), lambda i: (0,)),
        ],
        out_specs=pl.BlockSpec((BT, 3 * D), lambda i: (i, 0)),
        out_shape=jax.ShapeDtypeStruct((S, 3 * D), jnp.float32),
    )(x, g, b, w, bias)


def _k2_body(q_ref, k_ref, v_ref, o_ref):
    qi = pl.program_id(1)
    q = q_ref[0]
    k = k_ref[0]
    s = jax.lax.dot_general(q, k, (((1,), (1,)), ((), ())),
                            preferred_element_type=jnp.float32)
    s = s * (1.0 / np.sqrt(HD).astype(np.float32))
    row = qi * BQ + jax.lax.broadcasted_iota(jnp.int32, (BQ, S), 0)
    col = jax.lax.broadcasted_iota(jnp.int32, (BQ, S), 1)
    s = jnp.where(col <= row, s, jnp.finfo(jnp.float32).min)
    m = jnp.max(s, axis=-1, keepdims=True)
    e = jnp.exp(s - m)
    p = e / jnp.sum(e, axis=-1, keepdims=True)
    o_ref[0] = jax.lax.dot_general(p, v_ref[0], (((1,), (0,)), ((), ())),
                                   preferred_element_type=jnp.float32)


def _k2(q, k, v):
    return pl.pallas_call(
        _k2_body,
        grid=(H, S // BQ),
        in_specs=[
            pl.BlockSpec((1, BQ, HD), lambda h, i: (h, i, 0)),
            pl.BlockSpec((1, S, HD), lambda h, i: (h, 0, 0)),
            pl.BlockSpec((1, S, HD), lambda h, i: (h, 0, 0)),
        ],
        out_specs=pl.BlockSpec((1, BQ, HD), lambda h, i: (h, i, 0)),
        out_shape=jax.ShapeDtypeStruct((H, S, HD), jnp.float32),
    )(q, k, v)


def _k3_body(x_ref, attn_ref, wo_ref, bo_ref, g_ref, b_ref, wg_ref,
             h_ref, h2_ref, logits_ref, a1_ref, a2_ref, w1_ref, w2_ref):
    o = jnp.dot(attn_ref[...], wo_ref[...], preferred_element_type=jnp.float32) + bo_ref[...]
    h = o + x_ref[...]
    h_ref[...] = h
    h2 = _ln(h, g_ref[...], b_ref[...])
    h2_ref[...] = h2
    logits = jnp.dot(h2, wg_ref[...], preferred_element_type=jnp.float32)
    logits_ref[...] = logits
    lm = jnp.max(logits, axis=-1, keepdims=True)
    ex = jnp.exp(logits - lm)
    p = ex / jnp.sum(ex, axis=-1, keepdims=True)
    idx = jax.lax.broadcasted_iota(jnp.int32, (BT, E), 1)
    m1 = jnp.max(p, axis=-1, keepdims=True)
    a1 = jnp.min(jnp.where(p >= m1, idx, E), axis=-1, keepdims=True)
    p2 = jnp.where(idx == a1, -1.0, p)
    m2 = jnp.max(p2, axis=-1, keepdims=True)
    a2 = jnp.min(jnp.where(p2 >= m2, idx, E), axis=-1, keepdims=True)
    denom = m1 + m2
    a1_ref[...] = a1
    a2_ref[...] = a2
    w1_ref[...] = m1 / denom
    w2_ref[...] = m2 / denom


def _k3(x, attn, wo, bo, g, b, wg):
    n = S // BT
    return pl.pallas_call(
        _k3_body,
        grid=(n,),
        in_specs=[
            pl.BlockSpec((BT, D), lambda i: (i, 0)),
            pl.BlockSpec((BT, D), lambda i: (i, 0)),
            pl.BlockSpec((D, D), lambda i: (0, 0)),
            pl.BlockSpec((D,), lambda i: (0,)),
            pl.BlockSpec((D,), lambda i: (0,)),
            pl.BlockSpec((D,), lambda i: (0,)),
            pl.BlockSpec((D, E), lambda i: (0, 0)),
        ],
        out_specs=[
            pl.BlockSpec((BT, D), lambda i: (i, 0)),
            pl.BlockSpec((BT, D), lambda i: (i, 0)),
            pl.BlockSpec((BT, E), lambda i: (i, 0)),
            pl.BlockSpec((BT, 1), lambda i: (i, 0)),
            pl.BlockSpec((BT, 1), lambda i: (i, 0)),
            pl.BlockSpec((BT, 1), lambda i: (i, 0)),
            pl.BlockSpec((BT, 1), lambda i: (i, 0)),
        ],
        out_shape=[
            jax.ShapeDtypeStruct((S, D), jnp.float32),
            jax.ShapeDtypeStruct((S, D), jnp.float32),
            jax.ShapeDtypeStruct((S, E), jnp.float32),
            jax.ShapeDtypeStruct((S, 1), jnp.int32),
            jax.ShapeDtypeStruct((S, 1), jnp.int32),
            jax.ShapeDtypeStruct((S, 1), jnp.float32),
            jax.ShapeDtypeStruct((S, 1), jnp.float32),
        ],
    )(x, attn, wo, bo, g, b, wg)


def _gmm_body(s_ref, xs_ref, wfc_ref, bfc_ref, wpj_ref, bpj_ref, out_ref):
    i = pl.program_id(0)

    @pl.when(i < s_ref[0])
    def _():
        x = xs_ref[...].astype(jnp.bfloat16)
        hmid = jnp.dot(x, wfc_ref[0], preferred_element_type=jnp.float32) + bfc_ref[0, 0]
        hmid = _gelu_new(hmid).astype(jnp.bfloat16)
        out_ref[...] = jnp.dot(hmid, wpj_ref[0], preferred_element_type=jnp.float32) + bpj_ref[0, 0]


def _gmm(sref, xs, wfc, bfc, wpj, bpj):
    grid_spec = pltpu.PrefetchScalarGridSpec(
        num_scalar_prefetch=1,
        grid=(NT,),
        in_specs=[
            pl.BlockSpec((GT, D), lambda i, s: (i, 0)),
            pl.BlockSpec((1, D, INNER), lambda i, s: (s[1 + i], 0, 0)),
            pl.BlockSpec((1, 1, INNER), lambda i, s: (s[1 + i], 0, 0)),
            pl.BlockSpec((1, INNER, D), lambda i, s: (s[1 + i], 0, 0)),
            pl.BlockSpec((1, 1, D), lambda i, s: (s[1 + i], 0, 0)),
        ],
        out_specs=pl.BlockSpec((GT, D), lambda i, s: (i, 0)),
    )
    return pl.pallas_call(
        _gmm_body,
        grid_spec=grid_spec,
        out_shape=jax.ShapeDtypeStruct((P, D), jnp.float32),
    )(sref, xs, wfc, bfc, wpj, bpj)


def _k5_body(h_ref, y1_ref, y2_ref, w1_ref, w2_ref, out_ref):
    out_ref[...] = (h_ref[...] + w1_ref[...] * y1_ref[...]
                    + w2_ref[...] * y2_ref[...])


def _k5(h, y1, y2, w1, w2):
    n = S // BT
    return pl.pallas_call(
        _k5_body,
        grid=(n,),
        in_specs=[
            pl.BlockSpec((BT, D), lambda i: (i, 0)),
            pl.BlockSpec((BT, D), lambda i: (i, 0)),
            pl.BlockSpec((BT, D), lambda i: (i, 0)),
            pl.BlockSpec((BT, 1), lambda i: (i, 0)),
            pl.BlockSpec((BT, 1), lambda i: (i, 0)),
        ],
        out_specs=pl.BlockSpec((BT, D), lambda i: (i, 0)),
        out_shape=jax.ShapeDtypeStruct((S, D), jnp.float32),
    )(h, y1, y2, w1, w2)


def _dispatch_indices(a1, a2):
    e_flat = jnp.concatenate([a1, a2], axis=1).reshape(A)
    onehot = (e_flat[:, None] == jnp.arange(E, dtype=jnp.int32)[None, :]).astype(jnp.int32)
    ranks_incl = jnp.cumsum(onehot, axis=0)
    counts = ranks_incl[-1]
    rank = jnp.sum(onehot * ranks_incl, axis=1) - 1
    ntiles = (counts + GT - 1) // GT
    cum_tiles = jnp.cumsum(ntiles)
    num_active = cum_tiles[-1]
    tile_start = cum_tiles - ntiles
    pos = tile_start[e_flat] * GT + rank
    tid = jnp.arange(NT, dtype=jnp.int32)
    tile_e_raw = jnp.searchsorted(cum_tiles, tid, side="right").astype(jnp.int32)
    last_e = jnp.max(jnp.where(counts > 0, jnp.arange(E, dtype=jnp.int32), -1))
    tile_e = jnp.where(tid < num_active, tile_e_raw, last_e)
    sref = jnp.concatenate([num_active[None].astype(jnp.int32), tile_e])
    return sref, pos


def kernel(hidden_states, ln1_g, ln1_b, W_attn, b_attn, W_o, b_o,
           ln2_g, ln2_b, Wg, Wfc, bfc, Wpj, bpj):
    x2d = hidden_states.reshape(S, D)

    qkv = _k1(x2d, ln1_g, ln1_b, W_attn, b_attn)
    q = qkv[:, :D].reshape(S, H, HD).transpose(1, 0, 2)
    k = qkv[:, D:2 * D].reshape(S, H, HD).transpose(1, 0, 2)
    v = qkv[:, 2 * D:].reshape(S, H, HD).transpose(1, 0, 2)

    attn = _k2(q, k, v)
    attn2d = attn.transpose(1, 0, 2).reshape(S, D)

    h, h2, logits, a1, a2, w1, w2 = _k3(x2d, attn2d, W_o, b_o, ln2_g, ln2_b, Wg)

    sref, pos = _dispatch_indices(a1, a2)
    tok = jnp.arange(A, dtype=jnp.int32) // TOPK
    row_token = jnp.zeros((P,), jnp.int32).at[pos].set(tok)
    xs = jnp.take(h2, row_token, axis=0)

    ys = _gmm(sref, xs,
              Wfc.astype(jnp.bfloat16), bfc.reshape(E, 1, INNER),
              Wpj.astype(jnp.bfloat16), bpj.reshape(E, 1, D))

    pos2 = pos.reshape(S, TOPK)
    y1 = jnp.take(ys, pos2[:, 0], axis=0)
    y2 = jnp.take(ys, pos2[:, 1], axis=0)

    out2d = _k5(h, y1, y2, w1, w2)
    return out2d.reshape(1, S, D), logits
